# baseline (speedup 1.0000x reference)
import functools
import jax
import jax.numpy as jnp
from jax import lax
from jax.experimental import pallas as pl
from jax.experimental.pallas import tpu as pltpu, tpu_sc as plsc

NCH = 384
ROWS = 8 * NCH
D = 56 * 56
NW = 32
RPW = ROWS // NW
K = 16
NCHUNK = RPW // K
NGRP = D // 16


def _make_sc_kernel():
    mesh = plsc.VectorSubcoreMesh(core_axis_name="c", subcore_axis_name="s")

    @functools.partial(
        pl.kernel,
        mesh=mesh,
        out_type=jax.ShapeDtypeStruct((ROWS, D), jnp.float32),
        scratch_types=[
            pltpu.VMEM((K, D), jnp.float32),
            pltpu.VMEM((K, D), jnp.float32),
            pltpu.SemaphoreType.DMA,
            pltpu.SemaphoreType.DMA,
            pltpu.SemaphoreType.DMA,
            pltpu.SemaphoreType.DMA,
        ],
    )
    def k(x_hbm, o_hbm, buf0, buf1, gsem0, gsem1, ssem0, ssem1):
        wid = lax.axis_index("s") * 2 + lax.axis_index("c")
        base = wid * RPW
        b = base // NCH
        src0 = 2 * b * NCH + (NCH - 1) - base

        bufs = (buf0, buf1)
        gsems = (gsem0, gsem1)
        ssems = (ssem0, ssem1)

        def load(j):
            src_lo = pl.multiple_of(src0 - j * K - (K - 1), 8)
            return pltpu.make_async_copy(
                x_hbm.at[pl.ds(src_lo, K)], bufs[j % 2], gsems[j % 2]
            )

        def store(j):
            return pltpu.make_async_copy(
                bufs[j % 2],
                o_hbm.at[pl.ds(pl.multiple_of(base + j * K, 8), K)],
                ssems[j % 2],
            )

        def reverse_rows(buf):
            def body(c, _):
                col = pl.ds(c * 16, 16)
                for i in range(K // 2):
                    t0 = buf[i, col]
                    t1 = buf[K - 1 - i, col]
                    buf[i, col] = t1
                    buf[K - 1 - i, col] = t0
                return _

            lax.fori_loop(0, NGRP, body, None)

        load(0).start()
        for j in range(NCHUNK):
            if j + 1 < NCHUNK:
                if j >= 1:
                    store(j - 1).wait()
                load(j + 1).start()
            load(j).wait()
            reverse_rows(bufs[j % 2])
            store(j).start()
        store(NCHUNK - 2).wait()
        store(NCHUNK - 1).wait()

    return k


_sc_kernel = _make_sc_kernel()


def kernel(input):
    x = input.reshape(ROWS, D)
    out = _sc_kernel(x)
    return out.reshape(8, NCH, 56, 56)


# speedup vs baseline: 9.5834x; 9.5834x over previous
import jax
import jax.numpy as jnp
from jax.experimental import pallas as pl

NCH = 384
RB = 3136


def _rev_body(x_ref, o_ref):
    r = jax.lax.broadcasted_iota(jnp.int32, (NCH, NCH), 0)
    c = jax.lax.broadcasted_iota(jnp.int32, (NCH, NCH), 1)
    p = (r + c == NCH - 1).astype(jnp.float32)
    o_ref[...] = jax.lax.dot(
        x_ref[...], p, preferred_element_type=jnp.float32
    )


def kernel(input):
    b, c, h, w = input.shape
    xt = jnp.transpose(input, (0, 2, 3, 1)).reshape(b * h * w, c)
    n = b * h * w
    nblk = n // RB
    out = pl.pallas_call(
        _rev_body,
        grid=(nblk,),
        in_specs=[pl.BlockSpec((RB, c), lambda j: (j, 0))],
        out_specs=pl.BlockSpec((RB, c), lambda j: (j, 0)),
        out_shape=jax.ShapeDtypeStruct((n, c), input.dtype),
    )(xt)
    return jnp.transpose(out.reshape(b, h, w, c), (0, 3, 1, 2))


# speedup vs baseline: 10.0918x; 1.0530x over previous
import jax
import jax.numpy as jnp
from jax.experimental import pallas as pl

NCH = 384
RB = 3136


def _rev_body(x_ref, o_ref):
    r = jax.lax.broadcasted_iota(jnp.int32, (128, 128), 0)
    c = jax.lax.broadcasted_iota(jnp.int32, (128, 128), 1)
    p = (r + c == 127).astype(jnp.float32)
    for t in range(NCH // 128):
        s = NCH - 128 * (t + 1)
        o_ref[:, 128 * t : 128 * (t + 1)] = jax.lax.dot(
            x_ref[:, s : s + 128], p, preferred_element_type=jnp.float32
        )


def kernel(input):
    b, c, h, w = input.shape
    xt = jnp.transpose(input, (0, 2, 3, 1)).reshape(b * h * w, c)
    n = b * h * w
    nblk = n // RB
    out = pl.pallas_call(
        _rev_body,
        grid=(nblk,),
        in_specs=[pl.BlockSpec((RB, c), lambda j: (j, 0))],
        out_specs=pl.BlockSpec((RB, c), lambda j: (j, 0)),
        out_shape=jax.ShapeDtypeStruct((n, c), input.dtype),
    )(xt)
    return jnp.transpose(out.reshape(b, h, w, c), (0, 3, 1, 2))


# speedup vs baseline: 10.7699x; 1.0672x over previous
import jax
import jax.numpy as jnp
from jax.experimental import pallas as pl

NCH = 384
RB = 6272


def _rev_body(x_ref, o_ref):
    r = jax.lax.broadcasted_iota(jnp.int32, (128, 128), 0)
    c = jax.lax.broadcasted_iota(jnp.int32, (128, 128), 1)
    p = (r + c == 127).astype(jnp.float32)
    for t in range(NCH // 128):
        s = NCH - 128 * (t + 1)
        o_ref[:, 128 * t : 128 * (t + 1)] = jax.lax.dot(
            x_ref[:, s : s + 128], p, preferred_element_type=jnp.float32
        )


def kernel(input):
    b, c, h, w = input.shape
    xt = jnp.transpose(input, (0, 2, 3, 1)).reshape(b * h * w, c)
    n = b * h * w
    nblk = n // RB
    out = pl.pallas_call(
        _rev_body,
        grid=(nblk,),
        in_specs=[pl.BlockSpec((RB, c), lambda j: (j, 0))],
        out_specs=pl.BlockSpec((RB, c), lambda j: (j, 0)),
        out_shape=jax.ShapeDtypeStruct((n, c), input.dtype),
    )(xt)
    return jnp.transpose(out.reshape(b, h, w, c), (0, 3, 1, 2))
